# baseline (device time: 258226 ns/iter reference)
import jax
import jax.numpy as jnp
from jax import lax
from jax.experimental import pallas as pl
from jax.experimental.pallas import tpu as pltpu

B = 16
H = 16
D = 64
HD = H * D
KV_SHARD = 1024
SPLIT = 4
KV_CHUNK = KV_SHARD // SPLIT
SCALE = D ** -0.5


def _head_expand_mask():
    col = lax.broadcasted_iota(jnp.int32, (H, HD), 1)
    row = lax.broadcasted_iota(jnp.int32, (H, HD), 0)
    return (col // D == row).astype(jnp.float32)


def _compute_body(p_ref, q_ref, k_ref, v_ref, o_ref, m_ref, l_ref):
    del p_ref
    km = k_ref[...].reshape(KV_CHUNK * H, D)
    vm = v_ref[...].reshape(KV_CHUNK * H, D)
    qm = q_ref[0, 0]
    s2 = lax.dot_general(
        km, qm, (((1,), (1,)), ((), ())),
        preferred_element_type=jnp.float32,
    )
    row = lax.broadcasted_iota(jnp.int32, (KV_CHUNK * H, H), 0)
    col = lax.broadcasted_iota(jnp.int32, (KV_CHUNK * H, H), 1)
    s_m = jnp.where(row % H == col, s2 * SCALE, -jnp.inf)
    m = jnp.max(s_m, axis=0, keepdims=True)
    p = jnp.exp(s_m - m)
    l = jnp.sum(p, axis=0, keepdims=True)
    o = lax.dot_general(
        p, vm, (((0,), (0,)), ((), ())),
        preferred_element_type=jnp.float32,
    )
    o_ref[0, 0] = o
    m_ref[0] = m
    l_ref[0] = l


def _combine_body(
    o_in, m_in, l_in, out_ref,
    acc_o, acc_ml, recv_o, recv_ml,
    send_o_sem, recv_o_sem, send_ml_sem, recv_ml_sem,
):
    my = [lax.axis_index(a) for a in ("x", "y", "z")]
    peers = []
    for ax in range(3):
        pc = list(my)
        pc[ax] = 1 - pc[ax]
        peers.append(tuple(pc))

    barrier = pltpu.get_barrier_semaphore()
    for pc in peers:
        pl.semaphore_signal(
            barrier, inc=1, device_id=pc,
            device_id_type=pl.DeviceIdType.MESH,
        )
    pl.semaphore_wait(barrier, 3)

    acc_o[...] = o_in[...]
    acc_ml[0] = m_in[...]
    acc_ml[1] = l_in[...]

    E = _head_expand_mask()

    def expand(a):
        return lax.dot_general(
            a, E, (((1,), (0,)), ((), ())),
            preferred_element_type=jnp.float32,
        )

    for step, pc in enumerate(peers):
        rdma_o = pltpu.make_async_remote_copy(
            src_ref=acc_o, dst_ref=recv_o.at[step],
            send_sem=send_o_sem.at[step], recv_sem=recv_o_sem.at[step],
            device_id=pc, device_id_type=pl.DeviceIdType.MESH,
        )
        rdma_ml = pltpu.make_async_remote_copy(
            src_ref=acc_ml, dst_ref=recv_ml.at[step],
            send_sem=send_ml_sem.at[step], recv_sem=recv_ml_sem.at[step],
            device_id=pc, device_id_type=pl.DeviceIdType.MESH,
        )
        rdma_o.start()
        rdma_ml.start()
        rdma_o.wait()
        rdma_ml.wait()

        m_a = acc_ml[0]
        l_a = acc_ml[1]
        m_b = recv_ml[step, 0]
        l_b = recv_ml[step, 1]
        m_n = jnp.maximum(m_a, m_b)
        ea = jnp.exp(m_a - m_n)
        eb = jnp.exp(m_b - m_n)
        acc_o[...] = acc_o[...] * expand(ea) + recv_o[step] * expand(eb)
        acc_ml[0] = m_n
        acc_ml[1] = l_a * ea + l_b * eb

    out_ref[...] = acc_o[...] / expand(acc_ml[1])


def kernel(Q, K, V):
    p_idx = lax.axis_index("x") * 2 + lax.axis_index("z")
    p_arr = jnp.reshape(p_idx, (1,)).astype(jnp.int32)

    grid_spec = pltpu.PrefetchScalarGridSpec(
        num_scalar_prefetch=1,
        grid=(B,),
        in_specs=[
            pl.BlockSpec((1, 1, H, D), lambda b, p: (b, 0, 0, 0)),
            pl.BlockSpec((1, KV_CHUNK, H, D), lambda b, p: (b, p[0], 0, 0)),
            pl.BlockSpec((1, KV_CHUNK, H, D), lambda b, p: (b, p[0], 0, 0)),
        ],
        out_specs=[
            pl.BlockSpec((1, 1, H, D), lambda b, p: (b, 0, 0, 0)),
            pl.BlockSpec((1, 1, H), lambda b, p: (b, 0, 0)),
            pl.BlockSpec((1, 1, H), lambda b, p: (b, 0, 0)),
        ],
    )
    o_part, m_part, l_part = pl.pallas_call(
        _compute_body,
        grid_spec=grid_spec,
        out_shape=[
            jax.ShapeDtypeStruct((B, 1, H, D), jnp.float32),
            jax.ShapeDtypeStruct((B, 1, H), jnp.float32),
            jax.ShapeDtypeStruct((B, 1, H), jnp.float32),
        ],
    )(p_arr, Q, K, V)
    o_part = o_part.reshape(B, HD)
    m_part = m_part.reshape(B, H)
    l_part = l_part.reshape(B, H)

    out = pl.pallas_call(
        _combine_body,
        out_shape=jax.ShapeDtypeStruct((B, HD), jnp.float32),
        in_specs=[pl.BlockSpec(memory_space=pltpu.VMEM)] * 3,
        out_specs=pl.BlockSpec(memory_space=pltpu.VMEM),
        scratch_shapes=[
            pltpu.VMEM((B, HD), jnp.float32),
            pltpu.VMEM((2, B, H), jnp.float32),
            pltpu.VMEM((3, B, HD), jnp.float32),
            pltpu.VMEM((3, 2, B, H), jnp.float32),
            pltpu.SemaphoreType.DMA((3,)),
            pltpu.SemaphoreType.DMA((3,)),
            pltpu.SemaphoreType.DMA((3,)),
            pltpu.SemaphoreType.DMA((3,)),
        ],
        compiler_params=pltpu.CompilerParams(collective_id=0),
    )(o_part, m_part, l_part)

    return out.reshape(B, 1, H, D)


# device time: 33039 ns/iter; 7.8158x vs baseline; 7.8158x over previous
import jax
import jax.numpy as jnp
from jax import lax
from jax.experimental import pallas as pl
from jax.experimental.pallas import tpu as pltpu

B = 16
H = 16
D = 64
HD = H * D
KV_SHARD = 1024
SPLIT = 4
KV_CHUNK = KV_SHARD // SPLIT
SCALE = D ** -0.5


def _head_expand_mask():
    col = lax.broadcasted_iota(jnp.int32, (H, HD), 1)
    row = lax.broadcasted_iota(jnp.int32, (H, HD), 0)
    return (col // D == row).astype(jnp.float32)


def _compute_body(p_ref, q_ref, k_ref, v_ref, o_ref, m_ref, l_ref):
    del p_ref
    kt = k_ref[...].reshape(HD, KV_CHUNK)
    vt = v_ref[...].reshape(HD, KV_CHUNK)
    q = q_ref[0]
    E = _head_expand_mask()
    qbd = E * q
    s = lax.dot_general(
        qbd, kt, (((1,), (0,)), ((), ())),
        preferred_element_type=jnp.float32,
    ) * SCALE
    m = jnp.max(s, axis=1, keepdims=True)
    p = jnp.exp(s - m)
    l = jnp.sum(p, axis=1, keepdims=True)
    o_full = lax.dot_general(
        p, vt, (((1,), (1,)), ((), ())),
        preferred_element_type=jnp.float32,
    )
    o_ref[0] = jnp.sum(o_full * E, axis=0, keepdims=True)
    m_ref[0] = m
    l_ref[0] = l


def _combine_body(
    o_in, m_in, l_in, out_ref,
    acc_o, acc_ml, recv_o, recv_ml,
    send_o_sem, recv_o_sem, send_ml_sem, recv_ml_sem,
):
    my = [lax.axis_index(a) for a in ("x", "y", "z")]
    peers = []
    for ax in range(3):
        pc = list(my)
        pc[ax] = 1 - pc[ax]
        peers.append(tuple(pc))

    barrier = pltpu.get_barrier_semaphore()
    for pc in peers:
        pl.semaphore_signal(
            barrier, inc=1, device_id=pc,
            device_id_type=pl.DeviceIdType.MESH,
        )
    pl.semaphore_wait(barrier, 3)

    acc_o[...] = o_in[...]
    acc_ml[0] = m_in[...]
    acc_ml[1] = l_in[...]

    E = _head_expand_mask()

    def expand(a):
        return lax.dot_general(
            a, E, (((1,), (0,)), ((), ())),
            preferred_element_type=jnp.float32,
        )

    for step, pc in enumerate(peers):
        rdma_o = pltpu.make_async_remote_copy(
            src_ref=acc_o, dst_ref=recv_o.at[step],
            send_sem=send_o_sem.at[step], recv_sem=recv_o_sem.at[step],
            device_id=pc, device_id_type=pl.DeviceIdType.MESH,
        )
        rdma_ml = pltpu.make_async_remote_copy(
            src_ref=acc_ml, dst_ref=recv_ml.at[step],
            send_sem=send_ml_sem.at[step], recv_sem=recv_ml_sem.at[step],
            device_id=pc, device_id_type=pl.DeviceIdType.MESH,
        )
        rdma_o.start()
        rdma_ml.start()
        rdma_o.wait()
        rdma_ml.wait()

        m_a = acc_ml[0]
        l_a = acc_ml[1]
        m_b = recv_ml[step, 0]
        l_b = recv_ml[step, 1]
        m_n = jnp.maximum(m_a, m_b)
        ea = jnp.exp(m_a - m_n)
        eb = jnp.exp(m_b - m_n)
        acc_o[...] = acc_o[...] * expand(ea) + recv_o[step] * expand(eb)
        acc_ml[0] = m_n
        acc_ml[1] = l_a * ea + l_b * eb

    out_ref[...] = acc_o[...] / expand(acc_ml[1])


def kernel(Q, K, V):
    KT = jnp.transpose(K, (0, 2, 3, 1))
    VT = jnp.transpose(V, (0, 2, 3, 1))
    Q3 = Q.reshape(B, 1, HD)

    p_idx = lax.axis_index("x") * 2 + lax.axis_index("z")
    p_arr = jnp.reshape(p_idx, (1,)).astype(jnp.int32)

    grid_spec = pltpu.PrefetchScalarGridSpec(
        num_scalar_prefetch=1,
        grid=(B,),
        in_specs=[
            pl.BlockSpec((1, 1, HD), lambda b, p: (b, 0, 0)),
            pl.BlockSpec((1, H, D, KV_CHUNK), lambda b, p: (b, 0, 0, p[0])),
            pl.BlockSpec((1, H, D, KV_CHUNK), lambda b, p: (b, 0, 0, p[0])),
        ],
        out_specs=[
            pl.BlockSpec((1, 1, HD), lambda b, p: (b, 0, 0)),
            pl.BlockSpec((1, H, 1), lambda b, p: (b, 0, 0)),
            pl.BlockSpec((1, H, 1), lambda b, p: (b, 0, 0)),
        ],
    )
    o_part, m_part, l_part = pl.pallas_call(
        _compute_body,
        grid_spec=grid_spec,
        out_shape=[
            jax.ShapeDtypeStruct((B, 1, HD), jnp.float32),
            jax.ShapeDtypeStruct((B, H, 1), jnp.float32),
            jax.ShapeDtypeStruct((B, H, 1), jnp.float32),
        ],
    )(p_arr, Q3, KT, VT)
    o_part = o_part.reshape(B, HD)
    m_part = m_part.reshape(B, H)
    l_part = l_part.reshape(B, H)

    out = pl.pallas_call(
        _combine_body,
        out_shape=jax.ShapeDtypeStruct((B, HD), jnp.float32),
        in_specs=[pl.BlockSpec(memory_space=pltpu.VMEM)] * 3,
        out_specs=pl.BlockSpec(memory_space=pltpu.VMEM),
        scratch_shapes=[
            pltpu.VMEM((B, HD), jnp.float32),
            pltpu.VMEM((2, B, H), jnp.float32),
            pltpu.VMEM((3, B, HD), jnp.float32),
            pltpu.VMEM((3, 2, B, H), jnp.float32),
            pltpu.SemaphoreType.DMA((3,)),
            pltpu.SemaphoreType.DMA((3,)),
            pltpu.SemaphoreType.DMA((3,)),
            pltpu.SemaphoreType.DMA((3,)),
        ],
        compiler_params=pltpu.CompilerParams(collective_id=0),
    )(o_part, m_part, l_part)

    return out.reshape(B, 1, H, D)


# device time: 26932 ns/iter; 9.5881x vs baseline; 1.2268x over previous
import jax
import jax.numpy as jnp
from jax import lax
from jax.experimental import pallas as pl
from jax.experimental.pallas import tpu as pltpu

B = 16
H = 16
D = 64
HD = H * D
KV_SHARD = 1024
SPLIT = 4
KV_CHUNK = KV_SHARD // SPLIT
SCALE = D ** -0.5
GB = 4


def _head_expand_mask():
    col = lax.broadcasted_iota(jnp.int32, (H, HD), 1)
    row = lax.broadcasted_iota(jnp.int32, (H, HD), 0)
    return (col // D == row).astype(jnp.float32)


def _compute_body(p_ref, q_ref, k_ref, v_ref, o_ref, m_ref, l_ref):
    del p_ref
    E = _head_expand_mask()
    for g in range(GB):
        kt = k_ref[g].reshape(HD, KV_CHUNK)
        vt = v_ref[g].reshape(HD, KV_CHUNK)
        q = q_ref[g]
        qbd = E * q
        s = lax.dot_general(
            qbd, kt, (((1,), (0,)), ((), ())),
            preferred_element_type=jnp.float32,
        ) * SCALE
        m = jnp.max(s, axis=1, keepdims=True)
        p = jnp.exp(s - m)
        l = jnp.sum(p, axis=1, keepdims=True)
        o_full = lax.dot_general(
            p, vt, (((1,), (1,)), ((), ())),
            preferred_element_type=jnp.float32,
        )
        o_ref[g] = jnp.sum(o_full * E, axis=0, keepdims=True)
        m_ref[g] = m
        l_ref[g] = l


def _combine_body(
    o_in, m_in, l_in, out_ref,
    acc_o, acc_ml, recv_o, recv_ml,
    send_o_sem, recv_o_sem, send_ml_sem, recv_ml_sem,
):
    my = [lax.axis_index(a) for a in ("x", "y", "z")]
    peers = []
    for ax in range(3):
        pc = list(my)
        pc[ax] = 1 - pc[ax]
        peers.append(tuple(pc))

    barrier = pltpu.get_barrier_semaphore()
    for pc in peers:
        pl.semaphore_signal(
            barrier, inc=1, device_id=pc,
            device_id_type=pl.DeviceIdType.MESH,
        )
    pl.semaphore_wait(barrier, 3)

    acc_o[...] = o_in[...]
    acc_ml[0] = m_in[...]
    acc_ml[1] = l_in[...]

    E = _head_expand_mask()

    def expand(a):
        return lax.dot_general(
            a, E, (((1,), (0,)), ((), ())),
            preferred_element_type=jnp.float32,
        )

    for step, pc in enumerate(peers):
        rdma_o = pltpu.make_async_remote_copy(
            src_ref=acc_o, dst_ref=recv_o.at[step],
            send_sem=send_o_sem.at[step], recv_sem=recv_o_sem.at[step],
            device_id=pc, device_id_type=pl.DeviceIdType.MESH,
        )
        rdma_ml = pltpu.make_async_remote_copy(
            src_ref=acc_ml, dst_ref=recv_ml.at[step],
            send_sem=send_ml_sem.at[step], recv_sem=recv_ml_sem.at[step],
            device_id=pc, device_id_type=pl.DeviceIdType.MESH,
        )
        rdma_o.start()
        rdma_ml.start()
        rdma_o.wait()
        rdma_ml.wait()

        m_a = acc_ml[0]
        l_a = acc_ml[1]
        m_b = recv_ml[step, 0]
        l_b = recv_ml[step, 1]
        m_n = jnp.maximum(m_a, m_b)
        ea = jnp.exp(m_a - m_n)
        eb = jnp.exp(m_b - m_n)
        acc_o[...] = acc_o[...] * expand(ea) + recv_o[step] * expand(eb)
        acc_ml[0] = m_n
        acc_ml[1] = l_a * ea + l_b * eb

    out_ref[...] = acc_o[...] / expand(acc_ml[1])


def kernel(Q, K, V):
    KT = jnp.transpose(K, (0, 2, 3, 1))
    VT = jnp.transpose(V, (0, 2, 3, 1))
    Q3 = Q.reshape(B, 1, HD)

    p_idx = lax.axis_index("x") * 2 + lax.axis_index("z")
    p_arr = jnp.reshape(p_idx, (1,)).astype(jnp.int32)

    grid_spec = pltpu.PrefetchScalarGridSpec(
        num_scalar_prefetch=1,
        grid=(B // GB,),
        in_specs=[
            pl.BlockSpec((GB, 1, HD), lambda b, p: (b, 0, 0)),
            pl.BlockSpec((GB, H, D, KV_CHUNK), lambda b, p: (b, 0, 0, p[0])),
            pl.BlockSpec((GB, H, D, KV_CHUNK), lambda b, p: (b, 0, 0, p[0])),
        ],
        out_specs=[
            pl.BlockSpec((GB, 1, HD), lambda b, p: (b, 0, 0)),
            pl.BlockSpec((GB, H, 1), lambda b, p: (b, 0, 0)),
            pl.BlockSpec((GB, H, 1), lambda b, p: (b, 0, 0)),
        ],
    )
    o_part, m_part, l_part = pl.pallas_call(
        _compute_body,
        grid_spec=grid_spec,
        out_shape=[
            jax.ShapeDtypeStruct((B, 1, HD), jnp.float32),
            jax.ShapeDtypeStruct((B, H, 1), jnp.float32),
            jax.ShapeDtypeStruct((B, H, 1), jnp.float32),
        ],
    )(p_arr, Q3, KT, VT)
    o_part = o_part.reshape(B, HD)
    m_part = m_part.reshape(B, H)
    l_part = l_part.reshape(B, H)

    out = pl.pallas_call(
        _combine_body,
        out_shape=jax.ShapeDtypeStruct((B, HD), jnp.float32),
        in_specs=[pl.BlockSpec(memory_space=pltpu.VMEM)] * 3,
        out_specs=pl.BlockSpec(memory_space=pltpu.VMEM),
        scratch_shapes=[
            pltpu.VMEM((B, HD), jnp.float32),
            pltpu.VMEM((2, B, H), jnp.float32),
            pltpu.VMEM((3, B, HD), jnp.float32),
            pltpu.VMEM((3, 2, B, H), jnp.float32),
            pltpu.SemaphoreType.DMA((3,)),
            pltpu.SemaphoreType.DMA((3,)),
            pltpu.SemaphoreType.DMA((3,)),
            pltpu.SemaphoreType.DMA((3,)),
        ],
        compiler_params=pltpu.CompilerParams(collective_id=0),
    )(o_part, m_part, l_part)

    return out.reshape(B, 1, H, D)


# device time: 25801 ns/iter; 10.0084x vs baseline; 1.0438x over previous
import functools

import jax
import jax.numpy as jnp
from jax import lax
from jax.experimental import pallas as pl
from jax.experimental.pallas import tpu as pltpu

B = 16
H = 16
D = 64
HD = H * D
KV_SHARD = 1024
SPLIT = 4
KV_CHUNK = KV_SHARD // SPLIT
SCALE = D ** -0.5
GB = 4
NG = B // GB
HOPS = 3


def _head_expand_mask():
    col = lax.broadcasted_iota(jnp.int32, (H, HD), 1)
    row = lax.broadcasted_iota(jnp.int32, (H, HD), 0)
    return (col // D == row).astype(jnp.float32)


def _expand(a, E):
    return lax.dot_general(
        a, E, (((0,), (0,)), ((), ())), preferred_element_type=jnp.float32
    )


def _flash_partial(q, kt, vt, E):
    qbd = E * q
    s = lax.dot_general(
        qbd, kt, (((1,), (0,)), ((), ())),
        preferred_element_type=jnp.float32,
    ) * SCALE
    m = jnp.max(s, axis=1, keepdims=True)
    p = jnp.exp(s - m)
    l = jnp.sum(p, axis=1, keepdims=True)
    o_full = lax.dot_general(
        p, vt, (((1,), (1,)), ((), ())),
        preferred_element_type=jnp.float32,
    )
    o = jnp.sum(o_full * E, axis=0, keepdims=True)
    return o, m, l


def _fused_body(
    p_ref, q_ref, k_ref, v_ref, out_ref,
    acc_o, acc_ml, recv_o, recv_ml,
    send_o_sem, recv_o_sem, send_ml_sem, recv_ml_sem,
):
    del p_ref
    t = pl.program_id(0)
    my = [lax.axis_index(a) for a in ("x", "y", "z")]
    peers = []
    for ax in range(HOPS):
        pc = list(my)
        pc[ax] = 1 - pc[ax]
        peers.append(tuple(pc))
    E = _head_expand_mask()

    def hop_rdma(h, g):
        ro = pltpu.make_async_remote_copy(
            src_ref=acc_o.at[g], dst_ref=recv_o.at[h, g],
            send_sem=send_o_sem.at[h, g], recv_sem=recv_o_sem.at[h, g],
            device_id=peers[h], device_id_type=pl.DeviceIdType.MESH,
        )
        rml = pltpu.make_async_remote_copy(
            src_ref=acc_ml.at[g], dst_ref=recv_ml.at[h, g],
            send_sem=send_ml_sem.at[h, g], recv_sem=recv_ml_sem.at[h, g],
            device_id=peers[h], device_id_type=pl.DeviceIdType.MESH,
        )
        return ro, rml

    @pl.when(t == 0)
    def _():
        barrier = pltpu.get_barrier_semaphore()
        for pc in peers:
            pl.semaphore_signal(
                barrier, inc=1, device_id=pc,
                device_id_type=pl.DeviceIdType.MESH,
            )
        pl.semaphore_wait(barrier, HOPS)

    for g in range(NG):
        @pl.when(t == g)
        def _(g=g):
            for gi in range(GB):
                kt = k_ref[gi].reshape(HD, KV_CHUNK)
                vt = v_ref[gi].reshape(HD, KV_CHUNK)
                o, m, l = _flash_partial(q_ref[gi], kt, vt, E)
                acc_o[g, gi:gi + 1, :] = o
                acc_ml[g, 0, :, gi:gi + 1] = m
                acc_ml[g, 1, :, gi:gi + 1] = l
            ro, rml = hop_rdma(0, g)
            ro.start()
            rml.start()

    for h in range(HOPS):
        for g in range(NG):
            @pl.when(t == g + 1 + h)
            def _(h=h, g=g):
                ro, rml = hop_rdma(h, g)
                ro.wait()
                rml.wait()
                m_a = acc_ml[g, 0]
                l_a = acc_ml[g, 1]
                m_b = recv_ml[h, g, 0]
                l_b = recv_ml[h, g, 1]
                m_n = jnp.maximum(m_a, m_b)
                ea = jnp.exp(m_a - m_n)
                eb = jnp.exp(m_b - m_n)
                o_n = acc_o[g] * _expand(ea, E) + recv_o[h, g] * _expand(eb, E)
                l_n = l_a * ea + l_b * eb
                if h + 1 < HOPS:
                    acc_o[g] = o_n
                    acc_ml[g, 0] = m_n
                    acc_ml[g, 1] = l_n
                    ro2, rml2 = hop_rdma(h + 1, g)
                    ro2.start()
                    rml2.start()
                else:
                    out_ref[:, 0, :] = o_n / _expand(l_n, E)


def kernel(Q, K, V):
    KT = jnp.transpose(K, (0, 2, 3, 1))
    VT = jnp.transpose(V, (0, 2, 3, 1))
    Q3 = Q.reshape(B, 1, HD)

    p_idx = lax.axis_index("x") * 2 + lax.axis_index("z")
    p_arr = jnp.reshape(p_idx, (1,)).astype(jnp.int32)

    def in_idx(t, p):
        g = jnp.minimum(t, NG - 1)
        return g, 0, 0, p[0]

    def q_idx(t, p):
        return jnp.minimum(t, NG - 1), 0, 0

    def out_idx(t, p):
        return jnp.clip(t - HOPS, 0, NG - 1), 0, 0

    grid_spec = pltpu.PrefetchScalarGridSpec(
        num_scalar_prefetch=1,
        grid=(NG + HOPS,),
        in_specs=[
            pl.BlockSpec((GB, 1, HD), q_idx),
            pl.BlockSpec((GB, H, D, KV_CHUNK), in_idx),
            pl.BlockSpec((GB, H, D, KV_CHUNK), in_idx),
        ],
        out_specs=pl.BlockSpec((GB, 1, HD), out_idx),
        scratch_shapes=[
            pltpu.VMEM((NG, GB, HD), jnp.float32),
            pltpu.VMEM((NG, 2, H, GB), jnp.float32),
            pltpu.VMEM((HOPS, NG, GB, HD), jnp.float32),
            pltpu.VMEM((HOPS, NG, 2, H, GB), jnp.float32),
            pltpu.SemaphoreType.DMA((HOPS, NG)),
            pltpu.SemaphoreType.DMA((HOPS, NG)),
            pltpu.SemaphoreType.DMA((HOPS, NG)),
            pltpu.SemaphoreType.DMA((HOPS, NG)),
        ],
    )
    out = pl.pallas_call(
        _fused_body,
        grid_spec=grid_spec,
        out_shape=jax.ShapeDtypeStruct((B, 1, HD), jnp.float32),
        compiler_params=pltpu.CompilerParams(collective_id=0),
    )(p_arr, Q3, KT, VT)

    return out.reshape(B, 1, H, D)
